# Initial kernel scaffold; baseline (speedup 1.0000x reference)
import jax
import jax.numpy as jnp
from jax.experimental import pallas as pl


def kernel(x, proj, emb_weight):
    raise NotImplementedError("write your pallas kernel here")


# speedup vs baseline: 15.5380x; 15.5380x over previous
import numpy as np
import jax
import jax.numpy as jnp
from jax import lax
from jax.experimental import pallas as pl
from jax.experimental.pallas import tpu as pltpu

_INP_DIM = 128
_EMB_DIM = 64
_N_PROJ = 16
_NUM_BINS = 20
_NCOL = (_NUM_BINS + 1) * _N_PROJ
_BLK = 512


def _body(x_ref, proj_ref, exp_ref, lo_ref, hi_ref, emb_ref, o_ref):
    xb = x_ref[...]
    s = jnp.sum(xb * xb, axis=1, keepdims=True)
    n = jnp.sqrt(s)
    xn = xb / jnp.maximum(n, 1e-12)
    z = jnp.dot(xn, proj_ref[...])
    zrep = lax.dot(z, exp_ref[...], precision=lax.Precision.HIGHEST)
    oh = ((zrep > lo_ref[...]) & (zrep <= hi_ref[...])).astype(jnp.float32)
    o_ref[...] = lax.dot(oh, emb_ref[...], precision=lax.Precision.HIGHEST)


def kernel(x, proj, emb_weight):
    bs, seq_len, _ = x.shape
    rows = bs * seq_len
    xf = x.reshape(rows, _INP_DIM)

    resolution = 2.0 / float(_NUM_BINS)
    grid = jnp.linspace(-1.0, 1.0, _NUM_BINS + 1)[:-1] + 0.5 * resolution
    lo21 = jnp.concatenate([jnp.full((1,), -3.0, jnp.float32), grid])
    hi21 = jnp.concatenate([grid, jnp.full((1,), 3.0, jnp.float32)])
    lo = jnp.tile(lo21, (_N_PROJ,)).reshape(1, _NCOL)
    hi = jnp.tile(hi21, (_N_PROJ,)).reshape(1, _NCOL)
    expand = jnp.asarray(
        (np.arange(_NCOL)[None, :] // (_NUM_BINS + 1) == np.arange(_N_PROJ)[:, None])
        .astype(np.float32))

    nblk = rows // _BLK
    out = pl.pallas_call(
        _body,
        grid=(nblk,),
        in_specs=[
            pl.BlockSpec((_BLK, _INP_DIM), lambda i: (i, 0)),
            pl.BlockSpec((_INP_DIM, _N_PROJ), lambda i: (0, 0)),
            pl.BlockSpec((_N_PROJ, _NCOL), lambda i: (0, 0)),
            pl.BlockSpec((1, _NCOL), lambda i: (0, 0)),
            pl.BlockSpec((1, _NCOL), lambda i: (0, 0)),
            pl.BlockSpec((_NCOL, _EMB_DIM), lambda i: (0, 0)),
        ],
        out_specs=pl.BlockSpec((_BLK, _EMB_DIM), lambda i: (i, 0)),
        out_shape=jax.ShapeDtypeStruct((rows, _EMB_DIM), jnp.float32),
        compiler_params=pltpu.CompilerParams(
            dimension_semantics=("arbitrary",)),
    )(xf, proj, expand, lo, hi, emb_weight)
    return out.reshape(bs, seq_len, _EMB_DIM)
